# baseline (speedup 1.0000x reference)
import functools

import jax
import jax.numpy as jnp
from jax import lax
from jax.experimental import pallas as pl

N_NODES = 39
F_IN = 12
HID = 64
FC = 128
N_OUT = 5 * N_NODES

def _norm_adj_t(a, at):
    deg_row = jnp.sum(a, axis=0, keepdims=True)
    deg_col = jnp.sum(at, axis=1, keepdims=True)
    dr = 1.0 / jnp.sqrt(jnp.maximum(deg_row, 1e-12))
    dc = 1.0 / jnp.sqrt(jnp.maximum(deg_col, 1e-12))
    return dc * at * dr


def _mix3(mt, h3):
    return lax.dot_general(mt, h3, (((1,), (0,)), ((), ())),
                           preferred_element_type=jnp.float32)


def _fused(x_ref, aa_ref, w12_ref, bb_ref, w3_ref, wh_ref, o_ref, *, bs):
    bf16 = jnp.bfloat16
    mt = _norm_adj_t(aa_ref[0], aa_ref[1]).astype(bf16)
    w12 = w12_ref[...].astype(bf16)
    w1 = w12[:F_IN]
    w2 = w12[F_IN:F_IN + HID]
    bb = bb_ref[...]
    b1 = bb[:, 0:HID]
    b2 = bb[:, HID:2 * HID]
    b3 = bb[:, 2 * HID:2 * HID + FC]
    bh = bb[:, 2 * HID + FC:2 * HID + FC + N_OUT]

    xm = _mix3(mt, x_ref[...])
    xr = xm.reshape(N_NODES * bs, F_IN).astype(bf16)
    h1 = jnp.dot(xr, w1, preferred_element_type=jnp.float32)
    x1 = jax.nn.relu(h1 + b1).astype(bf16)
    h2 = jnp.dot(x1, w2, preferred_element_type=jnp.float32)
    h23 = h2.astype(bf16).reshape(N_NODES, bs, HID)
    x2m = _mix3(mt, h23)

    y = jnp.zeros((bs, FC), dtype=jnp.float32)
    for n in range(N_NODES):
        x2n = jnp.tanh(x2m[n] + b2)
        y = y + jnp.dot(x2n.astype(bf16), w3_ref[n].astype(bf16),
                        preferred_element_type=jnp.float32)
    y = jax.nn.relu(y + b3).astype(bf16)
    o_ref[...] = (jnp.dot(y, wh_ref[...].astype(bf16),
                          preferred_element_type=jnp.float32) + bh)


def _full(shape):
    return pl.BlockSpec(shape, lambda *_: tuple(0 for _ in shape))


def kernel(wav_input, graph_features, graph_input, gcn1_W, gcn1_b, gcn2_W,
           gcn2_b, fc1_W, fc1_b, pga_W, pga_b, pgv_W, pgv_b, sa03_W, sa03_b,
           sa10_W, sa10_b, sa30_W, sa30_b):
    del graph_features
    B = wav_input.shape[0]
    f32 = jnp.float32
    bf16 = jnp.bfloat16
    bs = 256

    xt = jnp.transpose(wav_input, (1, 0, 2)).astype(bf16)
    a = graph_input[0]
    aa = jnp.stack([a, a.T])
    w12 = jnp.concatenate([gcn1_W, gcn2_W], axis=0)
    bb = jnp.concatenate([gcn1_b, gcn2_b, fc1_b, pga_b, pgv_b, sa03_b,
                          sa10_b, sa30_b]).reshape(1, -1)
    w3 = fc1_W.reshape(N_NODES, HID, FC)
    wh = jnp.concatenate([pga_W, pgv_W, sa03_W, sa10_W, sa30_W], axis=1)

    out = pl.pallas_call(
        functools.partial(_fused, bs=bs),
        grid=(B // bs,),
        in_specs=[pl.BlockSpec((N_NODES, bs, F_IN), lambda i: (0, i, 0)),
                  _full((2, N_NODES, N_NODES)),
                  _full((F_IN + HID, HID)), _full((1, 2 * HID + FC + N_OUT)),
                  _full((N_NODES, HID, FC)), _full((FC, N_OUT))],
        out_specs=pl.BlockSpec((bs, N_OUT), lambda i: (i, 0)),
        out_shape=jax.ShapeDtypeStruct((B, N_OUT), f32),
    )(xt, aa, w12, bb, w3, wh)

    return (out[:, 0:39], out[:, 39:78], out[:, 78:117],
            out[:, 117:156], out[:, 156:195])


# speedup vs baseline: 1.0301x; 1.0301x over previous
import functools

import jax
import jax.numpy as jnp
from jax import lax
from jax.experimental import pallas as pl

N_NODES = 39
F_IN = 12
HID = 64
FC = 128
N_OUT = 5 * N_NODES

def _norm_adj_t(a, at):
    deg_row = jnp.sum(a, axis=0, keepdims=True)
    deg_col = jnp.sum(at, axis=1, keepdims=True)
    dr = 1.0 / jnp.sqrt(jnp.maximum(deg_row, 1e-12))
    dc = 1.0 / jnp.sqrt(jnp.maximum(deg_col, 1e-12))
    return dc * at * dr


def _mix3(mt, h3):
    return lax.dot_general(mt, h3, (((1,), (0,)), ((), ())),
                           preferred_element_type=jnp.float32)


def _fused(x_ref, aa_ref, w1_ref, b1_ref, w2_ref, b2_ref, w3_ref,
           b3_ref, wh_ref, bh_ref, o_ref, *, bs):
    bf16 = jnp.bfloat16
    mt = _norm_adj_t(aa_ref[0], aa_ref[1]).astype(bf16)
    w1 = w1_ref[...].astype(bf16)
    w2 = w2_ref[...].astype(bf16)

    xm = _mix3(mt, x_ref[...])
    xr = xm.reshape(N_NODES * bs, F_IN).astype(bf16)
    h1 = jnp.dot(xr, w1, preferred_element_type=jnp.float32)
    x1 = jax.nn.relu(h1 + b1_ref[...]).astype(bf16)
    h2 = jnp.dot(x1, w2, preferred_element_type=jnp.float32)
    h23 = h2.astype(bf16).reshape(N_NODES, bs, HID)
    x2m = _mix3(mt, h23)

    y = jnp.zeros((bs, FC), dtype=jnp.float32)
    for n in range(N_NODES):
        x2n = jnp.tanh(x2m[n] + b2_ref[...])
        y = y + jnp.dot(x2n.astype(bf16), w3_ref[n].astype(bf16),
                        preferred_element_type=jnp.float32)
    y = jax.nn.relu(y + b3_ref[...]).astype(bf16)
    o_ref[...] = (jnp.dot(y, wh_ref[...].astype(bf16),
                          preferred_element_type=jnp.float32) + bh_ref[...])


def _full(shape):
    return pl.BlockSpec(shape, lambda *_: tuple(0 for _ in shape))


def kernel(wav_input, graph_features, graph_input, gcn1_W, gcn1_b, gcn2_W,
           gcn2_b, fc1_W, fc1_b, pga_W, pga_b, pgv_W, pgv_b, sa03_W, sa03_b,
           sa10_W, sa10_b, sa30_W, sa30_b):
    del graph_features
    B = wav_input.shape[0]
    f32 = jnp.float32
    bf16 = jnp.bfloat16
    bs = 256

    xt = jnp.transpose(wav_input, (1, 0, 2)).astype(bf16)
    a = graph_input[0]
    aa = jnp.stack([a, a.T])
    b1 = gcn1_b.reshape(1, HID)
    b2 = gcn2_b.reshape(1, HID)
    w3 = fc1_W.reshape(N_NODES, HID, FC)
    b3 = fc1_b.reshape(1, FC)
    wh = jnp.concatenate([pga_W, pgv_W, sa03_W, sa10_W, sa30_W], axis=1)
    bh = jnp.concatenate([pga_b, pgv_b, sa03_b, sa10_b, sa30_b]).reshape(1, N_OUT)

    out = pl.pallas_call(
        functools.partial(_fused, bs=bs),
        grid=(B // bs,),
        in_specs=[pl.BlockSpec((N_NODES, bs, F_IN), lambda i: (0, i, 0)),
                  _full((2, N_NODES, N_NODES)),
                  _full((F_IN, HID)), _full((1, HID)),
                  _full((HID, HID)), _full((1, HID)),
                  _full((N_NODES, HID, FC)), _full((1, FC)),
                  _full((FC, N_OUT)), _full((1, N_OUT))],
        out_specs=pl.BlockSpec((bs, N_OUT), lambda i: (i, 0)),
        out_shape=jax.ShapeDtypeStruct((B, N_OUT), f32),
    )(xt, aa, gcn1_W, b1, gcn2_W, b2, w3, b3, wh, bh)

    return (out[:, 0:39], out[:, 39:78], out[:, 78:117],
            out[:, 117:156], out[:, 156:195])
